# baseline (speedup 1.0000x reference)
import jax
import jax.numpy as jnp
from jax.experimental import pallas as pl


def _lin(p, x):
    return x @ p["w"] + p["b"]


def _egnn_fwd(p, h, x, edge_index, edge_attr):
    row, col = edge_index[0], edge_index[1]
    n = h.shape[0]
    h = _lin(p["emb_in"], h)
    cnt = jnp.clip(jnp.zeros((n,), jnp.float32).at[row].add(1.0), 1.0)
    for lp in p["layers"]:
        xd = x[row] - x[col]
        r2 = jnp.sum(xd * xd, axis=1, keepdims=True)
        m = jax.nn.silu(_lin(lp["e1"], jnp.concatenate([h[row], h[col], r2, edge_attr], axis=1)))
        m = jax.nn.silu(_lin(lp["e2"], m))
        trans = xd * (jax.nn.silu(_lin(lp["c1"], m)) @ lp["c2w"])
        x = x + jnp.zeros_like(x).at[row].add(trans) / cnt[:, None]
        agg = jnp.zeros((n, m.shape[1]), m.dtype).at[row].add(m)
        h = h + _lin(lp["n2"], jax.nn.silu(_lin(lp["n1"], jnp.concatenate([h, agg], axis=1))))
    return _lin(p["emb_out"], h), x


def kernel(atm_node_feat, atm_coords, atm_edge_index, atm_edge_attr, res_node_feat, res_coords, res_edge_index, res_edge_attr, r2a, params):
    h = jax.nn.elu(_lin(params["res_lin1"], res_node_feat))
    h = _lin(params["res_lin2"], h)
    h, _ = _egnn_fwd(params["res_egnn"], h, res_coords, res_edge_index, res_edge_attr)
    h_resA = r2a @ h
    n0 = atm_node_feat
    h00 = _lin(params["lin00b"], jax.nn.elu(_lin(params["lin00"], n0[:, 1:22])))
    h01 = _lin(params["lin01b"], jax.nn.elu(_lin(params["lin01"], n0[:, 22:87])))
    proj = jnp.concatenate([n0[:, :1], h00, h01, n0[:, 87:]], axis=1)
    h_atm = jax.nn.elu(_lin(params["lin1_atm"], proj))
    h_cat = jnp.concatenate([h_resA, h_atm], axis=1)
    h_atm, _ = _egnn_fwd(params["atm_egnn"], h_cat, atm_coords, atm_edge_index, atm_edge_attr)
    pooled = jnp.mean(h_atm, axis=0, keepdims=True)
    z = jax.nn.gelu(_lin(params["cls1"], pooled), approximate=False)
    z = jax.nn.gelu(_lin(params["cls2"], z), approximate=False)
    return _lin(params["cls3"], z)


# speedup vs baseline: 4.4973x; 4.4973x over previous
import functools

import jax
import jax.numpy as jnp
from jax import lax
from jax.experimental import pallas as pl
from jax.experimental.pallas import tpu as pltpu
from jax.experimental.pallas import tpu_sc as plsc

NC = 2
NS = 16
NW = NC * NS
C = 128

_MESH = dict(core_axis_name="c", subcore_axis_name="s")


@functools.lru_cache(maxsize=None)
def _make_gather(E, D):
    Ew = E // NW
    assert Ew * NW == E
    nfull, rem = divmod(Ew, C)
    scratch = [pltpu.VMEM((C,), jnp.int32), pltpu.VMEM((C, D), jnp.float32)]
    if rem:
        scratch += [pltpu.VMEM((rem,), jnp.int32), pltpu.VMEM((rem, D), jnp.float32)]
    scratch += [pltpu.SemaphoreType.DMA]

    def body(table_hbm, idx_hbm, out_hbm, *s):
        if rem:
            idx_v, rows_v, idx_t, rows_t, sem = s
        else:
            idx_v, rows_v, sem = s
        wid = lax.axis_index("s") * NC + lax.axis_index("c")
        base = wid * Ew

        def step(g, carry):
            off = base + g * C
            pltpu.sync_copy(idx_hbm.at[pl.ds(off, C)], idx_v)
            pltpu.async_copy(table_hbm.at[idx_v], rows_v, sem).wait()
            pltpu.sync_copy(rows_v, out_hbm.at[pl.ds(off, C)])
            return carry

        lax.fori_loop(0, nfull, step, 0)
        if rem:
            off = base + nfull * C
            pltpu.sync_copy(idx_hbm.at[pl.ds(off, rem)], idx_t)
            pltpu.async_copy(table_hbm.at[idx_t], rows_t, sem).wait()
            pltpu.sync_copy(rows_t, out_hbm.at[pl.ds(off, rem)])

    return pl.kernel(
        body,
        out_type=jax.ShapeDtypeStruct((E, D), jnp.float32),
        mesh=plsc.VectorSubcoreMesh(**_MESH),
        scratch_types=scratch,
        compiler_params=pltpu.CompilerParams(use_tc_tiling_on_sc=False),
    )


@functools.lru_cache(maxsize=None)
def _make_scatter_add(E, Npad, D):
    Ew = E // NW
    assert Ew * NW == E
    nfull, rem = divmod(Ew, C)
    RPS = Npad // NS
    assert RPS * NS == Npad
    scratch = [pltpu.VMEM((C,), jnp.int32), pltpu.VMEM((C, D), jnp.float32)]
    if rem:
        scratch += [pltpu.VMEM((rem,), jnp.int32), pltpu.VMEM((rem, D), jnp.float32)]
    scratch += [pltpu.VMEM_SHARED((Npad, D), jnp.float32)]

    def body(vals_hbm, idx_hbm, zeros_hbm, out_hbm, *s):
        if rem:
            idx_v, vals_v, idx_t, vals_t, accum = s
        else:
            idx_v, vals_v, accum = s
        cid = lax.axis_index("c")
        sid = lax.axis_index("s")
        wid = sid * NC + cid
        base = wid * Ew
        pltpu.sync_copy(zeros_hbm, accum.at[pl.ds(sid * RPS, RPS)])
        plsc.subcore_barrier()

        def step(g, carry):
            off = base + g * C
            pltpu.sync_copy(idx_hbm.at[pl.ds(off, C)], idx_v)
            pltpu.sync_copy(vals_hbm.at[pl.ds(off, C)], vals_v)
            pltpu.sync_copy(vals_v, accum.at[idx_v], add=True)
            return carry

        lax.fori_loop(0, nfull, step, 0)
        if rem:
            off = base + nfull * C
            pltpu.sync_copy(idx_hbm.at[pl.ds(off, rem)], idx_t)
            pltpu.sync_copy(vals_hbm.at[pl.ds(off, rem)], vals_t)
            pltpu.sync_copy(vals_t, accum.at[idx_t], add=True)
        plsc.subcore_barrier()
        pltpu.sync_copy(accum.at[pl.ds(sid * RPS, RPS)],
                        out_hbm.at[cid, pl.ds(sid * RPS, RPS)])

    return pl.kernel(
        body,
        out_type=jax.ShapeDtypeStruct((NC, Npad, D), jnp.float32),
        mesh=plsc.VectorSubcoreMesh(**_MESH),
        scratch_types=scratch,
        compiler_params=pltpu.CompilerParams(use_tc_tiling_on_sc=False),
    )


def _sc_gather(table, idx, D):
    E = idx.shape[0]
    return _make_gather(E, D)(table, idx)


def _sc_scatter_add(vals, idx, Npad, D):
    E = idx.shape[0]
    zeros = jnp.zeros((Npad // NS, D), jnp.float32)
    return _make_scatter_add(E, Npad, D)(vals, idx, zeros)


def _lin(p, x):
    return x @ p["w"] + p["b"]


def _pad16(n):
    return (n + 15) // 16 * 16


def _egnn_fwd(p, h, x, edge_index, edge_attr):
    row, col = edge_index[0], edge_index[1]
    n = h.shape[0]
    npad = _pad16(n)
    h = _lin(p["emb_in"], h)
    cnt = None
    nl = len(p["layers"])
    for li, lp in enumerate(p["layers"]):
        last = li == nl - 1
        W1, b1 = lp["e1"]["w"], lp["e1"]["b"]
        pad = jnp.zeros((n, 48 - 35), jnp.float32)
        t_row = jnp.concatenate([h @ W1[:32] + b1, x, pad], axis=1)
        t_col = jnp.concatenate([h @ W1[32:64], -x, pad], axis=1)
        g = _sc_gather(t_row, row, 48) + _sc_gather(t_col, col, 48)
        xd = g[:, 32:35]
        r2 = jnp.sum(xd * xd, axis=1, keepdims=True)
        t1 = g[:, :32] + r2 * W1[64:65] + edge_attr @ W1[65:67]
        m = jax.nn.silu(t1)
        m = jax.nn.silu(_lin(lp["e2"], m))
        if not last:
            s = jax.nn.silu(_lin(lp["c1"], m)) @ lp["c2w"]
            ones = jnp.ones((m.shape[0], 1), jnp.float32)
            vals = jnp.concatenate([m, xd * s, ones,
                                    jnp.zeros((m.shape[0], 48 - 36), jnp.float32)], axis=1)
            parts = _sc_scatter_add(vals, row, npad, 48)
            S = (parts[0] + parts[1])[:n]
            agg = S[:, :32]
            if cnt is None:
                cnt = jnp.clip(S[:, 35], 1.0)
            x = x + S[:, 32:35] / cnt[:, None]
        else:
            parts = _sc_scatter_add(m, row, npad, 32)
            agg = (parts[0] + parts[1])[:n]
        h = h + _lin(lp["n2"], jax.nn.silu(_lin(lp["n1"], jnp.concatenate([h, agg], axis=1))))
    return _lin(p["emb_out"], h)


def kernel(atm_node_feat, atm_coords, atm_edge_index, atm_edge_attr, res_node_feat, res_coords, res_edge_index, res_edge_attr, r2a, params):
    h = jax.nn.elu(_lin(params["res_lin1"], res_node_feat))
    h = _lin(params["res_lin2"], h)
    h = _egnn_fwd(params["res_egnn"], h, res_coords, res_edge_index, res_edge_attr)
    h_resA = r2a @ h
    n0 = atm_node_feat
    h00 = _lin(params["lin00b"], jax.nn.elu(_lin(params["lin00"], n0[:, 1:22])))
    h01 = _lin(params["lin01b"], jax.nn.elu(_lin(params["lin01"], n0[:, 22:87])))
    proj = jnp.concatenate([n0[:, :1], h00, h01, n0[:, 87:]], axis=1)
    h_atm = jax.nn.elu(_lin(params["lin1_atm"], proj))
    h_cat = jnp.concatenate([h_resA, h_atm], axis=1)
    h_atm = _egnn_fwd(params["atm_egnn"], h_cat, atm_coords, atm_edge_index, atm_edge_attr)
    pooled = jnp.mean(h_atm, axis=0, keepdims=True)
    z = jax.nn.gelu(_lin(params["cls1"], pooled), approximate=False)
    z = jax.nn.gelu(_lin(params["cls2"], z), approximate=False)
    return _lin(params["cls3"], z)


# speedup vs baseline: 6.5706x; 1.4610x over previous
import functools

import jax
import jax.numpy as jnp
from jax import lax
from jax.experimental import pallas as pl
from jax.experimental.pallas import tpu as pltpu
from jax.experimental.pallas import tpu_sc as plsc

NC = 2
NS = 16
NW = NC * NS
C = 128

_MESH = dict(core_axis_name="c", subcore_axis_name="s")


def _row_add(bufr, bufc, outb, nrows, D):
    def rbody(i, carry):
        for j in range(D // 16):
            sl = pl.ds(j * 16, 16)
            outb[i, sl] = bufr[i, sl] + bufc[i, sl]
        return carry
    lax.fori_loop(0, nrows, rbody, 0)


@functools.lru_cache(maxsize=None)
def _make_gather2(E, D):
    Ew = E // NW
    assert Ew * NW == E
    nfull, rem = divmod(Ew, C)
    nsteady = nfull if nfull % 2 == 0 else nfull - 1
    scratch = []
    for _ in range(2):
        scratch += [pltpu.VMEM((C,), jnp.int32), pltpu.VMEM((C,), jnp.int32),
                    pltpu.VMEM((C, D), jnp.float32), pltpu.VMEM((C, D), jnp.float32),
                    pltpu.VMEM((C, D), jnp.float32),
                    pltpu.SemaphoreType.DMA, pltpu.SemaphoreType.DMA,
                    pltpu.SemaphoreType.DMA]
    if rem:
        scratch += [pltpu.VMEM((rem,), jnp.int32), pltpu.VMEM((rem,), jnp.int32),
                    pltpu.VMEM((rem, D), jnp.float32), pltpu.VMEM((rem, D), jnp.float32),
                    pltpu.VMEM((rem, D), jnp.float32)]

    def body(tr_hbm, tc_hbm, row_hbm, col_hbm, out_hbm, *s):
        bufs = [s[0:8], s[8:16]]
        tail = s[16:] if rem else None
        wid = lax.axis_index("s") * NC + lax.axis_index("c")
        base = wid * Ew

        def issue_idx(g, p):
            ir, ic = bufs[p][0], bufs[p][1]
            off = base + g * C
            pltpu.async_copy(row_hbm.at[pl.ds(off, C)], ir, bufs[p][5])
            pltpu.async_copy(col_hbm.at[pl.ds(off, C)], ic, bufs[p][5])

        def wait_idx(p):
            ir, ic = bufs[p][0], bufs[p][1]
            pltpu.make_async_copy(row_hbm.at[pl.ds(0, C)], ir, bufs[p][5]).wait()
            pltpu.make_async_copy(col_hbm.at[pl.ds(0, C)], ic, bufs[p][5]).wait()

        def issue_gather(p):
            ir, ic, br, bc = bufs[p][0], bufs[p][1], bufs[p][2], bufs[p][3]
            pltpu.async_copy(tr_hbm.at[ir], br, bufs[p][6])
            pltpu.async_copy(tc_hbm.at[ic], bc, bufs[p][6])

        def wait_gather(p):
            ir, ic, br, bc = bufs[p][0], bufs[p][1], bufs[p][2], bufs[p][3]
            pltpu.make_async_copy(tr_hbm.at[ir], br, bufs[p][6]).wait()
            pltpu.make_async_copy(tc_hbm.at[ic], bc, bufs[p][6]).wait()

        def issue_wb(g, p):
            off = base + g * C
            pltpu.async_copy(bufs[p][4], out_hbm.at[pl.ds(off, C)], bufs[p][7])

        def wait_wb(p):
            pltpu.make_async_copy(bufs[p][4], out_hbm.at[pl.ds(0, C)],
                                  bufs[p][7]).wait()

        if nsteady > 0:
            issue_idx(0, 0)
            if nsteady > 1:
                issue_idx(1, 1)
            wait_idx(0)
            issue_gather(0)

            def step(gp, carry):
                for p in (0, 1):
                    g = 2 * gp + p

                    @pl.when(g >= 2)
                    def _():
                        wait_wb(p)
                    wait_gather(p)

                    @pl.when(g + 2 < nsteady)
                    def _():
                        issue_idx(g + 2, p)
                    _row_add(bufs[p][2], bufs[p][3], bufs[p][4], C, D)
                    issue_wb(g, p)

                    @pl.when(g + 1 < nsteady)
                    def _():
                        wait_idx(1 - p)
                        issue_gather(1 - p)
                return carry

            lax.fori_loop(0, nsteady // 2, step, 0)
            wait_wb(0)
            if nsteady > 1:
                wait_wb(1)

        def sync_chunk(off, n, ir, ic, br, bc, ob):
            pltpu.sync_copy(row_hbm.at[pl.ds(off, n)], ir)
            pltpu.sync_copy(col_hbm.at[pl.ds(off, n)], ic)
            pltpu.async_copy(tr_hbm.at[ir], br, bufs[0][6])
            pltpu.async_copy(tc_hbm.at[ic], bc, bufs[0][6])
            pltpu.make_async_copy(tr_hbm.at[ir], br, bufs[0][6]).wait()
            pltpu.make_async_copy(tc_hbm.at[ic], bc, bufs[0][6]).wait()
            _row_add(br, bc, ob, n, D)
            pltpu.sync_copy(ob, out_hbm.at[pl.ds(off, n)])

        if nsteady < nfull:
            sync_chunk(base + nsteady * C, C, *bufs[0][:5])
        if rem:
            sync_chunk(base + nfull * C, rem, *tail)

    return pl.kernel(
        body,
        out_type=jax.ShapeDtypeStruct((E, D), jnp.float32),
        mesh=plsc.VectorSubcoreMesh(**_MESH),
        scratch_types=scratch,
        compiler_params=pltpu.CompilerParams(use_tc_tiling_on_sc=False),
    )


@functools.lru_cache(maxsize=None)
def _make_scatter_add(E, Npad, D):
    Ew = E // NW
    assert Ew * NW == E
    nfull, rem = divmod(Ew, C)
    nsteady = nfull if nfull % 2 == 0 else nfull - 1
    RPS = Npad // NS
    assert RPS * NS == Npad
    scratch = []
    for _ in range(2):
        scratch += [pltpu.VMEM((C,), jnp.int32), pltpu.VMEM((C, D), jnp.float32),
                    pltpu.SemaphoreType.DMA, pltpu.SemaphoreType.DMA]
    if rem:
        scratch += [pltpu.VMEM((rem,), jnp.int32), pltpu.VMEM((rem, D), jnp.float32)]
    scratch += [pltpu.VMEM_SHARED((Npad, D), jnp.float32)]

    def body(vals_hbm, idx_hbm, zeros_hbm, out_hbm, *s):
        bufs = [s[0:4], s[4:8]]
        tail = s[8:10] if rem else None
        accum = s[-1]
        cid = lax.axis_index("c")
        sid = lax.axis_index("s")
        wid = sid * NC + cid
        base = wid * Ew

        def issue_load(g, p):
            iv, vv = bufs[p][0], bufs[p][1]
            off = base + g * C
            pltpu.async_copy(idx_hbm.at[pl.ds(off, C)], iv, bufs[p][2])
            pltpu.async_copy(vals_hbm.at[pl.ds(off, C)], vv, bufs[p][2])

        def wait_load(p):
            iv, vv = bufs[p][0], bufs[p][1]
            pltpu.make_async_copy(idx_hbm.at[pl.ds(0, C)], iv, bufs[p][2]).wait()
            pltpu.make_async_copy(vals_hbm.at[pl.ds(0, C)], vv, bufs[p][2]).wait()

        def issue_scat(p):
            iv, vv = bufs[p][0], bufs[p][1]
            pltpu.async_copy(vv, accum.at[iv], bufs[p][3], add=True)

        def wait_scat(p):
            iv, vv = bufs[p][0], bufs[p][1]
            pltpu.make_async_copy(vv, accum.at[iv], bufs[p][3]).wait()

        if nsteady > 0:
            issue_load(0, 0)
        pltpu.sync_copy(zeros_hbm, accum.at[pl.ds(sid * RPS, RPS)])
        plsc.subcore_barrier()

        if nsteady > 0:
            def step(gp, carry):
                for p in (0, 1):
                    g = 2 * gp + p
                    wait_load(p)
                    issue_scat(p)

                    @pl.when(g + 1 < nsteady)
                    def _():
                        @pl.when(g >= 1)
                        def _():
                            wait_scat(1 - p)
                        issue_load(g + 1, 1 - p)
                return carry

            lax.fori_loop(0, nsteady // 2, step, 0)
            wait_scat((nsteady - 1) % 2)

        def sync_chunk(off, n, iv, vv):
            pltpu.sync_copy(idx_hbm.at[pl.ds(off, n)], iv)
            pltpu.sync_copy(vals_hbm.at[pl.ds(off, n)], vv)
            pltpu.sync_copy(vv, accum.at[iv], add=True)

        if nsteady < nfull:
            sync_chunk(base + nsteady * C, C, bufs[0][0], bufs[0][1])
        if rem:
            sync_chunk(base + nfull * C, rem, *tail)
        plsc.subcore_barrier()
        pltpu.sync_copy(accum.at[pl.ds(sid * RPS, RPS)],
                        out_hbm.at[cid, pl.ds(sid * RPS, RPS)])

    return pl.kernel(
        body,
        out_type=jax.ShapeDtypeStruct((NC, Npad, D), jnp.float32),
        mesh=plsc.VectorSubcoreMesh(**_MESH),
        scratch_types=scratch,
        compiler_params=pltpu.CompilerParams(use_tc_tiling_on_sc=False),
    )


def _sc_gather2(t_row, t_col, row, col, D):
    E = row.shape[0]
    return _make_gather2(E, D)(t_row, t_col, row, col)


def _sc_scatter_add(vals, idx, Npad, D):
    E = idx.shape[0]
    zeros = jnp.zeros((Npad // NS, D), jnp.float32)
    return _make_scatter_add(E, Npad, D)(vals, idx, zeros)


def _lin(p, x):
    return x @ p["w"] + p["b"]


def _pad16(n):
    return (n + 15) // 16 * 16


def _egnn_fwd(p, h, x, edge_index, edge_attr):
    row, col = edge_index[0], edge_index[1]
    n = h.shape[0]
    npad = _pad16(n)
    h = _lin(p["emb_in"], h)
    cnt = None
    nl = len(p["layers"])
    for li, lp in enumerate(p["layers"]):
        last = li == nl - 1
        W1, b1 = lp["e1"]["w"], lp["e1"]["b"]
        pad = jnp.zeros((n, 48 - 35), jnp.float32)
        t_row = jnp.concatenate([h @ W1[:32] + b1, x, pad], axis=1)
        t_col = jnp.concatenate([h @ W1[32:64], -x, pad], axis=1)
        g = _sc_gather2(t_row, t_col, row, col, 48)
        xd = g[:, 32:35]
        r2 = jnp.sum(xd * xd, axis=1, keepdims=True)
        t1 = g[:, :32] + r2 * W1[64:65] + edge_attr @ W1[65:67]
        m = jax.nn.silu(t1)
        m = jax.nn.silu(_lin(lp["e2"], m))
        if not last:
            s = jax.nn.silu(_lin(lp["c1"], m)) @ lp["c2w"]
            ones = jnp.ones((m.shape[0], 1), jnp.float32)
            vals = jnp.concatenate([m, xd * s, ones,
                                    jnp.zeros((m.shape[0], 48 - 36), jnp.float32)], axis=1)
            parts = _sc_scatter_add(vals, row, npad, 48)
            S = (parts[0] + parts[1])[:n]
            agg = S[:, :32]
            if cnt is None:
                cnt = jnp.clip(S[:, 35], 1.0)
            x = x + S[:, 32:35] / cnt[:, None]
        else:
            parts = _sc_scatter_add(m, row, npad, 32)
            agg = (parts[0] + parts[1])[:n]
        h = h + _lin(lp["n2"], jax.nn.silu(_lin(lp["n1"], jnp.concatenate([h, agg], axis=1))))
    return _lin(p["emb_out"], h)


def kernel(atm_node_feat, atm_coords, atm_edge_index, atm_edge_attr, res_node_feat, res_coords, res_edge_index, res_edge_attr, r2a, params):
    h = jax.nn.elu(_lin(params["res_lin1"], res_node_feat))
    h = _lin(params["res_lin2"], h)
    h = _egnn_fwd(params["res_egnn"], h, res_coords, res_edge_index, res_edge_attr)
    h_resA = r2a @ h
    n0 = atm_node_feat
    h00 = _lin(params["lin00b"], jax.nn.elu(_lin(params["lin00"], n0[:, 1:22])))
    h01 = _lin(params["lin01b"], jax.nn.elu(_lin(params["lin01"], n0[:, 22:87])))
    proj = jnp.concatenate([n0[:, :1], h00, h01, n0[:, 87:]], axis=1)
    h_atm = jax.nn.elu(_lin(params["lin1_atm"], proj))
    h_cat = jnp.concatenate([h_resA, h_atm], axis=1)
    h_atm = _egnn_fwd(params["atm_egnn"], h_cat, atm_coords, atm_edge_index, atm_edge_attr)
    pooled = jnp.mean(h_atm, axis=0, keepdims=True)
    z = jax.nn.gelu(_lin(params["cls1"], pooled), approximate=False)
    z = jax.nn.gelu(_lin(params["cls2"], z), approximate=False)
    return _lin(params["cls3"], z)
